# baseline (speedup 1.0000x reference)
import functools

import jax
import jax.numpy as jnp
from jax import lax
from jax.experimental import pallas as pl
from jax.experimental.pallas import tpu as pltpu
from jax.experimental.pallas import tpu_sc as plsc

NUM_EMBEDDINGS = 1000000
EMBEDDING_DIM = 64
BATCH = 4096
SEQ = 200

_info = plsc.get_sparse_core_info()
NC, NS = _info.num_cores, _info.num_subcores
NW = NC * NS

LANES = 128
NTILE_FULL = NUM_EMBEDDINGS // LANES
TAIL = NUM_EMBEDDINGS - NTILE_FULL * LANES
D_STEPS = (NTILE_FULL + NW - 1) // NW

BLK = BATCH // NW
TPOS = SEQ * 8 * NW
OUT_ROWS = TPOS * 8
IDXW = 133
TILW = 133

_mesh = plsc.VectorSubcoreMesh(core_axis_name="c", subcore_axis_name="s")


def _make_depad():
    @functools.partial(
        pl.kernel,
        mesh=_mesh,
        compiler_params=pltpu.CompilerParams(
            use_tc_tiling_on_sc=True, needs_layout_passes=False
        ),
        out_type=jax.ShapeDtypeStruct((NUM_EMBEDDINGS, LANES), jnp.float32),
        scratch_types=[
            pltpu.VMEM((2, EMBEDDING_DIM, LANES), jnp.float32),
            pltpu.VMEM((2, LANES, 130), jnp.float32),
            pltpu.VMEM((TAIL, LANES), jnp.float32),
        ]
        + [pltpu.SemaphoreType.DMA] * 4,
    )
    def depad_kernel(wt_hbm, aux_hbm, wpad_hbm, in_v, out_v, aux_v, *sems):
        in_sem = sems[:2]
        out_sem = sems[2:4]
        wid = lax.axis_index("s") * NC + lax.axis_index("c")
        iota = lax.iota(jnp.int32, 16)

        def in_copy(k, b):
            ti = k * NW + wid
            return pltpu.make_async_copy(
                wt_hbm.at[:, pl.ds(ti * LANES, LANES)], in_v.at[b], in_sem[b]
            )

        def out_copy(k, b):
            ti = k * NW + wid
            return pltpu.make_async_copy(
                out_v.at[b, :, pl.ds(0, LANES)],
                wpad_hbm.at[pl.ds(ti * LANES, LANES)],
                out_sem[b],
            )

        in_copy(0, 0).start()

        def step(k, b, have_next):
            in_copy(k, b).wait()

            @pl.when(have_next)
            def _prefetch():
                in_copy(k + 1, 1 - b).start()

            @pl.when(k >= 2)
            def _drain():
                out_copy(k - 2, b).wait()

            def tr_block(cb, _):
                for cj in range(8):
                    c = cb * 8 + cj
                    cols = jnp.full((16,), c, jnp.int32)
                    for lb in range(8):
                        v = in_v[b, c, pl.ds(lb * 16, 16)]
                        plsc.store_scatter(out_v.at[b], [lb * 16 + iota, cols], v)
                return _

            lax.fori_loop(0, 8, tr_block, 0, unroll=False)
            out_copy(k, b).start()

        def body(kk, _):
            for j in range(2):
                k = kk * 2 + j
                ti = k * NW + wid

                @pl.when(ti < NTILE_FULL)
                def _do():
                    nxt = (k + 1) * NW + wid
                    step(k, j, nxt < NTILE_FULL)

            return _

        lax.fori_loop(0, (D_STEPS + 1) // 2, body, 0, unroll=False)

        n_k = (NTILE_FULL - wid + NW - 1) // NW
        for b in range(2):
            for which in (1, 2):
                k_last = n_k - which

                @pl.when((k_last >= 0) & (k_last % 2 == b))
                def _fd():
                    out_copy(k_last, b).wait()

        @pl.when(wid == 0)
        def _tail():
            pltpu.sync_copy(aux_hbm, aux_v)
            pltpu.sync_copy(aux_v, wpad_hbm.at[pl.ds(NTILE_FULL * LANES, TAIL)])

    return depad_kernel


def _make_gather():
    @functools.partial(
        pl.kernel,
        mesh=_mesh,
        compiler_params=pltpu.CompilerParams(
            use_tc_tiling_on_sc=False, needs_layout_passes=False
        ),
        out_type=jax.ShapeDtypeStruct((OUT_ROWS, LANES), jnp.float32),
        scratch_types=[
            pltpu.VMEM((BLK, SEQ), jnp.int32),
            pltpu.VMEM((SEQ, IDXW), jnp.int32),
            pltpu.VMEM((2, BLK, LANES), jnp.float32),
            pltpu.VMEM((2, EMBEDDING_DIM, TILW), jnp.float32),
        ]
        + [pltpu.SemaphoreType.DMA] * 4,
    )
    def gather_kernel(idx_hbm, wpad_hbm, out_hbm, idx_v, idx_t, rows_v, tile_v, *sems):
        g_sem = sems[:2]
        w_sem = sems[2:4]
        wid = lax.axis_index("s") * NC + lax.axis_index("c")
        wb = wid
        iota = lax.iota(jnp.int32, 16)

        pltpu.sync_copy(idx_hbm.at[pl.ds(wb * BLK, BLK)], idx_v)
        s_starts = [16 * t for t in range(SEQ // 16)] + [SEQ - 16]

        def idtr(j, _):
            cols = jnp.full((16,), j, jnp.int32)
            for s0 in s_starts:
                v = idx_v[j, pl.ds(s0, 16)]
                plsc.store_scatter(idx_t, [s0 + iota, cols], v)
            return _

        lax.fori_loop(0, BLK, idtr, 0, unroll=False)

        def g_copy(s, b):
            return pltpu.make_async_copy(
                wpad_hbm.at[idx_t.at[s, pl.ds(0, BLK)]], rows_v.at[b], g_sem[b]
            )

        def w_copy(s, b, cb):
            base = ((s * 8 + cb) * NW + wb) * 8
            return pltpu.make_async_copy(
                tile_v.at[b, pl.ds(8 * cb, 8), pl.ds(0, LANES)],
                out_hbm.at[pl.ds(base, 8)],
                w_sem[b],
            )

        g_copy(0, 0).start()

        def step(s, b):
            g_copy(s, b).wait()

            @pl.when(s + 1 < SEQ)
            def _prefetch():
                g_copy(s + 1, 1 - b).start()

            @pl.when(s >= 2)
            def _drain():
                for cb in range(8):
                    w_copy(s - 2, b, cb).wait()

            def tr(jj, _):
                for j2 in range(2):
                    j = jj * 2 + j2
                    cols = jnp.full((16,), j, jnp.int32)
                    for c0 in range(0, EMBEDDING_DIM, 16):
                        v = rows_v[b, j, pl.ds(c0, 16)]
                        plsc.store_scatter(tile_v.at[b], [c0 + iota, cols], v)
                return _

            lax.fori_loop(0, BLK // 2, tr, 0, unroll=False)
            for cb in range(8):
                w_copy(s, b, cb).start()

        def body(s2, _):
            for par in range(2):
                step(s2 * 2 + par, par)
            return _

        lax.fori_loop(0, SEQ // 2, body, 0, unroll=False)

        for b in range(2):
            for cb in range(8):
                w_copy(SEQ - 2 + b, b, cb).wait()

    return gather_kernel


_depad = _make_depad()
_gather = _make_gather()


@jax.jit
def kernel(input_ids, weight):
    w_t = jnp.swapaxes(weight, 0, 1)
    aux = jnp.pad(weight[NTILE_FULL * LANES :, :], ((0, 0), (0, LANES - EMBEDDING_DIM)))
    wpad = _depad(w_t, aux)
    out5 = _gather(input_ids, wpad)
    return (
        out5.reshape(SEQ, 8, NW, 8, LANES)
        .transpose(2, 4, 0, 1, 3)
        .reshape(BATCH, SEQ, EMBEDDING_DIM)
    )


# speedup vs baseline: 1.0006x; 1.0006x over previous
import functools

import jax
import jax.numpy as jnp
from jax import lax
from jax.experimental import pallas as pl
from jax.experimental.pallas import tpu as pltpu
from jax.experimental.pallas import tpu_sc as plsc

NUM_EMBEDDINGS = 1000000
EMBEDDING_DIM = 64
BATCH = 4096
SEQ = 200

_info = plsc.get_sparse_core_info()
NC, NS = _info.num_cores, _info.num_subcores
NW = NC * NS

LANES = 128
NTILE_FULL = NUM_EMBEDDINGS // LANES
TAIL = NUM_EMBEDDINGS - NTILE_FULL * LANES
D_STEPS = (NTILE_FULL + NW - 1) // NW

BLK = BATCH // NW
TPOS = SEQ * 8 * NW
OUT_ROWS = TPOS * 8
IDXW = 133
TILW = 133

_mesh = plsc.VectorSubcoreMesh(core_axis_name="c", subcore_axis_name="s")


def _make_depad():
    @functools.partial(
        pl.kernel,
        mesh=_mesh,
        compiler_params=pltpu.CompilerParams(
            use_tc_tiling_on_sc=True, needs_layout_passes=False
        ),
        out_type=jax.ShapeDtypeStruct((NUM_EMBEDDINGS, LANES), jnp.float32),
        scratch_types=[
            pltpu.VMEM((2, EMBEDDING_DIM, LANES), jnp.float32),
            pltpu.VMEM((2, LANES, 129), jnp.float32),
            pltpu.VMEM((TAIL, LANES), jnp.float32),
        ]
        + [pltpu.SemaphoreType.DMA] * 4,
    )
    def depad_kernel(wt_hbm, aux_hbm, wpad_hbm, in_v, out_v, aux_v, *sems):
        in_sem = sems[:2]
        out_sem = sems[2:4]
        wid = lax.axis_index("s") * NC + lax.axis_index("c")
        iota = lax.iota(jnp.int32, 16)

        def in_copy(k, b):
            ti = k * NW + wid
            return pltpu.make_async_copy(
                wt_hbm.at[:, pl.ds(ti * LANES, LANES)], in_v.at[b], in_sem[b]
            )

        def out_copy(k, b):
            ti = k * NW + wid
            return pltpu.make_async_copy(
                out_v.at[b, :, pl.ds(0, LANES)],
                wpad_hbm.at[pl.ds(ti * LANES, LANES)],
                out_sem[b],
            )

        in_copy(0, 0).start()
        row_idx = [lb * 16 + iota for lb in range(8)]

        def step(k, b, have_next):
            in_copy(k, b).wait()

            @pl.when(have_next)
            def _prefetch():
                in_copy(k + 1, 1 - b).start()

            @pl.when(k >= 2)
            def _drain():
                out_copy(k - 2, b).wait()

            def tr_block(cb, _):
                for cj in range(8):
                    c = cb * 8 + cj
                    cols = jnp.full((16,), c, jnp.int32)
                    for lb in range(8):
                        v = in_v[b, c, pl.ds(lb * 16, 16)]
                        plsc.store_scatter(out_v.at[b], [row_idx[lb], cols], v)
                return _

            lax.fori_loop(0, 8, tr_block, 0, unroll=False)
            out_copy(k, b).start()

        def body(kk, _):
            for j in range(2):
                k = kk * 2 + j
                ti = k * NW + wid

                @pl.when(ti < NTILE_FULL)
                def _do():
                    nxt = (k + 1) * NW + wid
                    step(k, j, nxt < NTILE_FULL)

            return _

        lax.fori_loop(0, (D_STEPS + 1) // 2, body, 0, unroll=False)

        n_k = (NTILE_FULL - wid + NW - 1) // NW
        for b in range(2):
            for which in (1, 2):
                k_last = n_k - which

                @pl.when((k_last >= 0) & (k_last % 2 == b))
                def _fd():
                    out_copy(k_last, b).wait()

        @pl.when(wid == 0)
        def _tail():
            pltpu.sync_copy(aux_hbm, aux_v)
            pltpu.sync_copy(aux_v, wpad_hbm.at[pl.ds(NTILE_FULL * LANES, TAIL)])

    return depad_kernel


def _make_gather():
    @functools.partial(
        pl.kernel,
        mesh=_mesh,
        compiler_params=pltpu.CompilerParams(
            use_tc_tiling_on_sc=False, needs_layout_passes=False
        ),
        out_type=jax.ShapeDtypeStruct((OUT_ROWS, LANES), jnp.float32),
        scratch_types=[
            pltpu.VMEM((BLK, SEQ), jnp.int32),
            pltpu.VMEM((SEQ, IDXW), jnp.int32),
            pltpu.VMEM((2, BLK, LANES), jnp.float32),
            pltpu.VMEM((2, EMBEDDING_DIM, TILW), jnp.float32),
        ]
        + [pltpu.SemaphoreType.DMA] * 4,
    )
    def gather_kernel(idx_hbm, wpad_hbm, out_hbm, idx_v, idx_t, rows_v, tile_v, *sems):
        g_sem = sems[:2]
        w_sem = sems[2:4]
        wid = lax.axis_index("s") * NC + lax.axis_index("c")
        wb = wid
        iota = lax.iota(jnp.int32, 16)

        pltpu.sync_copy(idx_hbm.at[pl.ds(wb * BLK, BLK)], idx_v)
        s_starts = [16 * t for t in range(SEQ // 16)] + [SEQ - 16]
        s_idx = {s0: s0 + iota for s0 in s_starts}
        c_idx = [c0 + iota for c0 in range(0, EMBEDDING_DIM, 16)]

        def idtr(j, _):
            cols = jnp.full((16,), j, jnp.int32)
            for s0 in s_starts:
                v = idx_v[j, pl.ds(s0, 16)]
                plsc.store_scatter(idx_t, [s_idx[s0], cols], v)
            return _

        lax.fori_loop(0, BLK, idtr, 0, unroll=False)

        def g_copy(s, b):
            return pltpu.make_async_copy(
                wpad_hbm.at[idx_t.at[s, pl.ds(0, BLK)]], rows_v.at[b], g_sem[b]
            )

        def w_copy(s, b, cb):
            base = ((s * 8 + cb) * NW + wb) * 8
            return pltpu.make_async_copy(
                tile_v.at[b, pl.ds(8 * cb, 8), pl.ds(0, LANES)],
                out_hbm.at[pl.ds(base, 8)],
                w_sem[b],
            )

        g_copy(0, 0).start()

        def step(s, b):
            g_copy(s, b).wait()

            @pl.when(s + 1 < SEQ)
            def _prefetch():
                g_copy(s + 1, 1 - b).start()

            @pl.when(s >= 2)
            def _drain():
                for cb in range(8):
                    w_copy(s - 2, b, cb).wait()

            def tr(jj, _):
                for j2 in range(2):
                    j = jj * 2 + j2
                    cols = jnp.full((16,), j, jnp.int32)
                    for ci, c0 in enumerate(range(0, EMBEDDING_DIM, 16)):
                        v = rows_v[b, j, pl.ds(c0, 16)]
                        plsc.store_scatter(tile_v.at[b], [c_idx[ci], cols], v)
                return _

            lax.fori_loop(0, BLK // 2, tr, 0, unroll=False)
            for cb in range(8):
                w_copy(s, b, cb).start()

        def body(s2, _):
            for par in range(2):
                step(s2 * 2 + par, par)
            return _

        lax.fori_loop(0, SEQ // 2, body, 0, unroll=False)

        for b in range(2):
            for cb in range(8):
                w_copy(SEQ - 2 + b, b, cb).wait()

    return gather_kernel


_depad = _make_depad()
_gather = _make_gather()


@jax.jit
def kernel(input_ids, weight):
    w_t = jnp.swapaxes(weight, 0, 1)
    aux = jnp.pad(weight[NTILE_FULL * LANES :, :], ((0, 0), (0, LANES - EMBEDDING_DIM)))
    wpad = _depad(w_t, aux)
    out5 = _gather(input_ids, wpad)
    return (
        out5.reshape(SEQ, 8, NW, 8, LANES)
        .transpose(2, 4, 0, 1, 3)
        .reshape(BATCH, SEQ, EMBEDDING_DIM)
    )


# speedup vs baseline: 1.6171x; 1.6160x over previous
import functools

import jax
import jax.numpy as jnp
from jax import lax
from jax.experimental import pallas as pl
from jax.experimental.pallas import tpu as pltpu
from jax.experimental.pallas import tpu_sc as plsc

NUM_EMBEDDINGS = 1000000
EMBEDDING_DIM = 64
BATCH = 4096
SEQ = 200

_info = plsc.get_sparse_core_info()
NC, NS = _info.num_cores, _info.num_subcores
NW = NC * NS

LANES = 128
NTILE_FULL = NUM_EMBEDDINGS // LANES
TAIL = NUM_EMBEDDINGS - NTILE_FULL * LANES
D_STEPS = (NTILE_FULL + NW - 1) // NW

BLK = BATCH // NW
TPOS = SEQ * 8 * NW
OUT_ROWS = TPOS * 8
IDXW = 133
TILW = 133

_mesh = plsc.VectorSubcoreMesh(core_axis_name="c", subcore_axis_name="s")


def _make_depad():
    @functools.partial(
        pl.kernel,
        mesh=_mesh,
        compiler_params=pltpu.CompilerParams(
            use_tc_tiling_on_sc=True, needs_layout_passes=False
        ),
        out_type=jax.ShapeDtypeStruct((NUM_EMBEDDINGS, LANES), jnp.float32),
        scratch_types=[
            pltpu.VMEM((2, EMBEDDING_DIM, LANES), jnp.float32),
            pltpu.VMEM((2, LANES, 129), jnp.float32),
            pltpu.VMEM((TAIL, LANES), jnp.float32),
        ]
        + [pltpu.SemaphoreType.DMA] * 4,
    )
    def depad_kernel(wt_hbm, aux_hbm, wpad_hbm, in_v, out_v, aux_v, *sems):
        in_sem = sems[:2]
        out_sem = sems[2:4]
        wid = lax.axis_index("s") * NC + lax.axis_index("c")
        iota = lax.iota(jnp.int32, 16)

        def in_copy(k, b):
            ti = k * NW + wid
            return pltpu.make_async_copy(
                wt_hbm.at[:, pl.ds(ti * LANES, LANES)], in_v.at[b], in_sem[b]
            )

        def out_copy(k, b):
            ti = k * NW + wid
            return pltpu.make_async_copy(
                out_v.at[b, :, pl.ds(0, LANES)],
                wpad_hbm.at[pl.ds(ti * LANES, LANES)],
                out_sem[b],
            )

        in_copy(0, 0).start()
        row_idx = [lb * 16 + iota for lb in range(8)]

        def step(k, b, have_next):
            in_copy(k, b).wait()

            @pl.when(have_next)
            def _prefetch():
                in_copy(k + 1, 1 - b).start()

            @pl.when(k >= 2)
            def _drain():
                out_copy(k - 2, b).wait()

            def tr_block(cb, _):
                for cj in range(8):
                    c = cb * 8 + cj
                    cols = jnp.full((16,), c, jnp.int32)
                    for lb in range(8):
                        v = in_v[b, c, pl.ds(lb * 16, 16)]
                        plsc.store_scatter(out_v.at[b], [row_idx[lb], cols], v)
                return _

            lax.fori_loop(0, 8, tr_block, 0, unroll=False)
            out_copy(k, b).start()

        def body(kk, _):
            for j in range(2):
                k = kk * 2 + j
                ti = k * NW + wid

                @pl.when(ti < NTILE_FULL)
                def _do():
                    nxt = (k + 1) * NW + wid
                    step(k, j, nxt < NTILE_FULL)

            return _

        lax.fori_loop(0, (D_STEPS + 1) // 2, body, 0, unroll=False)

        n_k = (NTILE_FULL - wid + NW - 1) // NW
        for b in range(2):
            for which in (1, 2):
                k_last = n_k - which

                @pl.when((k_last >= 0) & (k_last % 2 == b))
                def _fd():
                    out_copy(k_last, b).wait()

        @pl.when(wid == 0)
        def _tail():
            pltpu.sync_copy(aux_hbm, aux_v)
            pltpu.sync_copy(aux_v, wpad_hbm.at[pl.ds(NTILE_FULL * LANES, TAIL)])

    return depad_kernel


def _make_gather():
    @functools.partial(
        pl.kernel,
        mesh=_mesh,
        compiler_params=pltpu.CompilerParams(
            use_tc_tiling_on_sc=False, needs_layout_passes=False
        ),
        out_type=jax.ShapeDtypeStruct((OUT_ROWS, LANES), jnp.float32),
        scratch_types=[
            pltpu.VMEM((BLK, SEQ), jnp.int32),
            pltpu.VMEM((SEQ, IDXW), jnp.int32),
            pltpu.VMEM((2, BLK, EMBEDDING_DIM), jnp.float32),
            pltpu.VMEM((2, EMBEDDING_DIM, TILW), jnp.float32),
        ]
        + [pltpu.SemaphoreType.DMA] * 4,
    )
    def gather_kernel(idx_hbm, wpad_hbm, out_hbm, idx_v, idx_t, rows_v, tile_v, *sems):
        g_sem = sems[:2]
        w_sem = sems[2:4]
        wid = lax.axis_index("s") * NC + lax.axis_index("c")
        wb = wid
        iota = lax.iota(jnp.int32, 16)

        pltpu.sync_copy(idx_hbm.at[pl.ds(wb * BLK, BLK)], idx_v)
        s_starts = [16 * t for t in range(SEQ // 16)] + [SEQ - 16]
        s_idx = {s0: s0 + iota for s0 in s_starts}
        c_idx = [c0 + iota for c0 in range(0, EMBEDDING_DIM, 16)]

        def idtr(j, _):
            cols = jnp.full((16,), j, jnp.int32)
            for s0 in s_starts:
                v = idx_v[j, pl.ds(s0, 16)]
                plsc.store_scatter(idx_t, [s_idx[s0], cols], v)
            return _

        lax.fori_loop(0, BLK, idtr, 0, unroll=False)

        def g_copy(s, b):
            return pltpu.make_async_copy(
                wpad_hbm.at[idx_t.at[s, pl.ds(0, BLK)]], rows_v.at[b], g_sem[b]
            )

        def w_copy(s, b, cb):
            base = ((s * 8 + cb) * NW + wb) * 8
            return pltpu.make_async_copy(
                tile_v.at[b, pl.ds(8 * cb, 8), pl.ds(0, LANES)],
                out_hbm.at[pl.ds(base, 8)],
                w_sem[b],
            )

        g_copy(0, 0).start()

        def step(s, b):
            g_copy(s, b).wait()

            @pl.when(s + 1 < SEQ)
            def _prefetch():
                g_copy(s + 1, 1 - b).start()

            @pl.when(s >= 2)
            def _drain():
                for cb in range(8):
                    w_copy(s - 2, b, cb).wait()

            def tr(jj, _):
                for j2 in range(2):
                    j = jj * 2 + j2
                    cols = jnp.full((16,), j, jnp.int32)
                    for ci, c0 in enumerate(range(0, EMBEDDING_DIM, 16)):
                        v = rows_v[b, j, pl.ds(c0, 16)]
                        plsc.store_scatter(tile_v.at[b], [c_idx[ci], cols], v)
                return _

            lax.fori_loop(0, BLK // 2, tr, 0, unroll=False)
            for cb in range(8):
                w_copy(s, b, cb).start()

        def body(s2, _):
            for par in range(2):
                step(s2 * 2 + par, par)
            return _

        lax.fori_loop(0, SEQ // 2, body, 0, unroll=False)

        for b in range(2):
            for cb in range(8):
                w_copy(SEQ - 2 + b, b, cb).wait()

    return gather_kernel


_gather = _make_gather()


@jax.jit
def kernel(input_ids, weight):
    out5 = _gather(input_ids, weight)
    return (
        out5.reshape(SEQ, 8, NW, 8, LANES)
        .transpose(2, 4, 0, 1, 3)
        .reshape(BATCH, SEQ, EMBEDDING_DIM)
    )


# speedup vs baseline: 1.6187x; 1.0010x over previous
import functools

import jax
import jax.numpy as jnp
from jax import lax
from jax.experimental import pallas as pl
from jax.experimental.pallas import tpu as pltpu
from jax.experimental.pallas import tpu_sc as plsc

NUM_EMBEDDINGS = 1000000
EMBEDDING_DIM = 64
BATCH = 4096
SEQ = 200

_info = plsc.get_sparse_core_info()
NC, NS = _info.num_cores, _info.num_subcores
NW = NC * NS

LANES = 128
BLK = BATCH // NW
TPOS = SEQ * 8 * NW
OUT_ROWS = TPOS * 8
IDXW = 133
TILW = 133

_mesh = plsc.VectorSubcoreMesh(core_axis_name="c", subcore_axis_name="s")


def _make_gather():
    @functools.partial(
        pl.kernel,
        mesh=_mesh,
        compiler_params=pltpu.CompilerParams(
            use_tc_tiling_on_sc=False, needs_layout_passes=False
        ),
        out_type=jax.ShapeDtypeStruct((OUT_ROWS, LANES), jnp.float32),
        scratch_types=[
            pltpu.VMEM((BLK, SEQ), jnp.int32),
            pltpu.VMEM((SEQ, IDXW), jnp.int32),
            pltpu.VMEM((2, BLK, EMBEDDING_DIM), jnp.float32),
            pltpu.VMEM((2, EMBEDDING_DIM, TILW), jnp.float32),
        ]
        + [pltpu.SemaphoreType.DMA] * 4,
    )
    def gather_kernel(idx_hbm, wpad_hbm, out_hbm, idx_v, idx_t, rows_v, tile_v, *sems):
        g_sem = sems[:2]
        w_sem = sems[2:4]
        wid = lax.axis_index("s") * NC + lax.axis_index("c")
        wb = wid
        iota = lax.iota(jnp.int32, 16)

        pltpu.sync_copy(idx_hbm.at[pl.ds(wb * BLK, BLK)], idx_v)
        s_starts = [16 * t for t in range(SEQ // 16)] + [SEQ - 16]
        s_idx = {s0: s0 + iota for s0 in s_starts}
        c_idx = [c0 + iota for c0 in range(0, EMBEDDING_DIM, 16)]

        def idtr(j, _):
            cols = jnp.full((16,), j, jnp.int32)
            for s0 in s_starts:
                v = idx_v[j, pl.ds(s0, 16)]
                plsc.store_scatter(idx_t, [s_idx[s0], cols], v)
            return _

        lax.fori_loop(0, BLK, idtr, 0, unroll=False)

        def g_copy(s, b):
            return pltpu.make_async_copy(
                wpad_hbm.at[idx_t.at[s, pl.ds(0, BLK)]], rows_v.at[b], g_sem[b]
            )

        def w_copy(s, b, cb):
            base = ((s * 8 + cb) * NW + wb) * 8
            return pltpu.make_async_copy(
                tile_v.at[b, pl.ds(8 * cb, 8), pl.ds(0, LANES)],
                out_hbm.at[pl.ds(base, 8)],
                w_sem[b],
            )

        g_copy(0, 0).start()

        def step(s, b):
            g_copy(s, b).wait()

            @pl.when(s + 1 < SEQ)
            def _prefetch():
                g_copy(s + 1, 1 - b).start()

            @pl.when(s >= 2)
            def _drain():
                for cb in range(8):
                    w_copy(s - 2, b, cb).wait()

            def tr(jj, _):
                for j2 in range(2):
                    j = jj * 2 + j2
                    cols = jnp.full((16,), j, jnp.int32)
                    for ci, c0 in enumerate(range(0, EMBEDDING_DIM, 16)):
                        v = rows_v[b, j, pl.ds(c0, 16)]
                        plsc.store_scatter(tile_v.at[b], [c_idx[ci], cols], v)
                return _

            lax.fori_loop(0, BLK // 2, tr, 0, unroll=False)
            for cb in range(8):
                w_copy(s, b, cb).start()

        def body(s2, _):
            for par in range(2):
                step(s2 * 2 + par, par)
            return _

        lax.fori_loop(0, SEQ // 2, body, 0, unroll=False)

        for b in range(2):
            for cb in range(8):
                w_copy(SEQ - 2 + b, b, cb).wait()

    return gather_kernel


_gather = _make_gather()


@jax.jit
def kernel(input_ids, weight):
    out5 = _gather(input_ids, weight)
    return (
        out5.reshape(SEQ, 8, NW, 8, LANES)
        .transpose(2, 4, 0, 1, 3)
        .reshape(BATCH, SEQ, EMBEDDING_DIM)
    )
